# baseline (speedup 1.0000x reference)
import functools

import jax
import jax.numpy as jnp
from jax import lax
from jax.experimental import pallas as pl
from jax.experimental.pallas import tpu as pltpu
from jax.experimental.pallas import tpu_sc as plsc

N_NODES = 10000
IN_CH = 256
HID = 128

NP = 10240
N_TILES = 32
N_CHUNKS = 40
CHUNK = 128
EP = N_TILES * N_CHUNKS * CHUNK
N_CROWS = EP // CHUNK
SEG_C0 = 40
SEG_C1 = 40
IDX_ROWS = 1280
ROWS_PER_TILE = NP // 16
N_CNT = N_TILES


def _make_segsum(with_cnt):
  mesh = plsc.VectorSubcoreMesh(core_axis_name="c", subcore_axis_name="s")

  dst_rows = 8 if with_cnt else SEG_C0
  out_type = [jax.ShapeDtypeStruct((2, NP, HID), jnp.float32)]
  scratch = [
      pltpu.VMEM((SEG_C0, CHUNK), jnp.int32),
      pltpu.VMEM((dst_rows, CHUNK), jnp.int32),
      pltpu.VMEM((CHUNK, HID), jnp.float32),
      pltpu.VMEM((CHUNK, HID), jnp.float32),
      pltpu.VMEM_SHARED((NP, HID), jnp.float32),
      pltpu.SemaphoreType.DMA,
      pltpu.SemaphoreType.DMA,
  ]
  if with_cnt:
    out_type.append(jax.ShapeDtypeStruct((N_TILES, NP), jnp.float32))
    scratch += [
        pltpu.VMEM((NP,), jnp.float32),
    ]

  def body(*refs):
    if with_cnt:
      (y_hbm, srci_hbm, dsti_hbm, zf_hbm, s_out, cnt_out,
       srci_v, dsti_v, rows0_v, rows1_v, acc_sh, sem0, sem1, hist_v) = refs
    else:
      (y_hbm, srci_hbm, dsti_hbm, zf_hbm, s_out,
       srci_v, dsti_v, rows0_v, rows1_v, acc_sh, sem0, sem1) = refs
    bufs = ((rows0_v, sem0), (rows1_v, sem1))

    c = lax.axis_index("c")
    s = lax.axis_index("s")
    r0 = s * ROWS_PER_TILE
    off = jnp.where(c == 0, s * SEG_C0, 16 * SEG_C0 + s * SEG_C1)
    n = jnp.where(c == 0, SEG_C0, SEG_C1)

    pltpu.sync_copy(zf_hbm.at[pl.ds(r0, ROWS_PER_TILE)],
                    acc_sh.at[pl.ds(r0, ROWS_PER_TILE)])
    ones16 = jnp.full((16,), 1.0, jnp.float32)
    zeros16 = jnp.zeros((16,), jnp.float32)
    if with_cnt:
      def zero_row(i, carry):
        for k in range(8):
          hist_v[pl.ds(i * 128 + k * 16, 16)] = zeros16
        return carry

      lax.fori_loop(0, NP // 128, zero_row, 0)
    plsc.subcore_barrier()

    pltpu.sync_copy(srci_hbm.at[pl.ds(off, SEG_C0)], srci_v)
    if not with_cnt:
      pltpu.sync_copy(dsti_hbm.at[pl.ds(off, SEG_C0)], dsti_v)

    for k, (buf, sem) in enumerate(bufs):
      pltpu.async_copy(y_hbm.at[srci_v.at[k]], buf, sem)

    if with_cnt:
      def window_step(w, carry):
        pltpu.sync_copy(dsti_hbm.at[pl.ds(off + w * 8, 8)], dsti_v)
        for jj in range(8):
          buf, sem = bufs[jj % 2]
          j = w * 8 + jj
          pltpu.make_async_copy(y_hbm.at[srci_v.at[j]], buf, sem).wait()
          pltpu.sync_copy(buf, acc_sh.at[dsti_v.at[jj]], add=True)
          nxt = jnp.minimum(j + 2, n - 1)
          pltpu.async_copy(y_hbm.at[srci_v.at[nxt]], buf, sem)
          for k8 in range(8):
            d = dsti_v[jj, pl.ds(k8 * 16, 16)]
            plsc.addupdate_scatter(hist_v, [d], ones16)
        return carry

      lax.fori_loop(0, n // 8, window_step, 0)
    else:
      def chunk_step(i, carry):
        for k, (buf, sem) in enumerate(bufs):
          j = 2 * i + k
          pltpu.make_async_copy(y_hbm.at[srci_v.at[j]], buf, sem).wait()
          pltpu.sync_copy(buf, acc_sh.at[dsti_v.at[j]], add=True)
          nxt = jnp.minimum(j + 2, n - 1)
          pltpu.async_copy(y_hbm.at[srci_v.at[nxt]], buf, sem)
        return carry

      lax.fori_loop(0, n // 2, chunk_step, 0)
    for buf, sem in bufs:
      pltpu.make_async_copy(y_hbm.at[srci_v.at[0]], buf, sem).wait()
    plsc.subcore_barrier()

    pltpu.sync_copy(acc_sh.at[pl.ds(r0, ROWS_PER_TILE)],
                    s_out.at[c, pl.ds(r0, ROWS_PER_TILE)])
    if with_cnt:
      pltpu.sync_copy(hist_v, cnt_out.at[c * 16 + s])

  params = pltpu.CompilerParams(needs_layout_passes=False) if with_cnt else None
  return pl.kernel(body, mesh=mesh, out_type=out_type, scratch_types=scratch,
                   compiler_params=params)


_segsum_cnt = _make_segsum(True)
_segsum = _make_segsum(False)


_BM = 2000


def _tc_proj2(xp, WlT, WrT, b):
  M, K = xp.shape
  N = WlT.shape[1]

  def body(x_ref, wl_ref, wr_ref, b_ref, y_ref, z_ref):
    x = x_ref[...]
    y_ref[...] = jnp.dot(x, wl_ref[...], preferred_element_type=jnp.float32,
                     precision=lax.Precision.HIGHEST)
    z_ref[...] = (jnp.dot(x, wr_ref[...], preferred_element_type=jnp.float32,
                     precision=lax.Precision.HIGHEST)
                  + b_ref[...])

  return pl.pallas_call(
      body,
      grid=(M // _BM,),
      in_specs=[
          pl.BlockSpec((_BM, K), lambda i: (i, 0)),
          pl.BlockSpec((K, N), lambda i: (0, 0)),
          pl.BlockSpec((K, N), lambda i: (0, 0)),
          pl.BlockSpec((1, N), lambda i: (0, 0)),
      ],
      out_specs=[
          pl.BlockSpec((_BM, N), lambda i: (i, 0)),
          pl.BlockSpec((_BM, N), lambda i: (i, 0)),
      ],
      out_shape=[
          jax.ShapeDtypeStruct((M, N), jnp.float32),
          jax.ShapeDtypeStruct((M, N), jnp.float32),
      ],
  )(xp, WlT, WrT, b)


def _tc_combine_proj2(s_pair, cnt_t, z, WlT, WrT, b):
  N = WlT.shape[1]

  def body(sa_ref, sb_ref, c_ref, z_ref, wl_ref, wr_ref, b_ref,
           y_ref, z2_ref):
    ssum = sa_ref[0] + sb_ref[0]
    cnt = jnp.sum(c_ref[...], axis=1, keepdims=True)
    mean = ssum / jnp.maximum(cnt, 1.0)
    h = jnp.maximum(mean + z_ref[...], 0.0)
    y_ref[...] = jnp.dot(h, wl_ref[...], preferred_element_type=jnp.float32,
                     precision=lax.Precision.HIGHEST)
    z2_ref[...] = (jnp.dot(h, wr_ref[...], preferred_element_type=jnp.float32,
                     precision=lax.Precision.HIGHEST)
                   + b_ref[...])

  return pl.pallas_call(
      body,
      grid=(N_NODES // _BM,),
      in_specs=[
          pl.BlockSpec((1, _BM, HID), lambda i: (0, i, 0)),
          pl.BlockSpec((1, _BM, HID), lambda i: (1, i, 0)),
          pl.BlockSpec((_BM, N_TILES), lambda i: (i, 0)),
          pl.BlockSpec((_BM, HID), lambda i: (i, 0)),
          pl.BlockSpec((HID, N), lambda i: (0, 0)),
          pl.BlockSpec((HID, N), lambda i: (0, 0)),
          pl.BlockSpec((1, N), lambda i: (0, 0)),
      ],
      out_specs=[
          pl.BlockSpec((_BM, N), lambda i: (i, 0)),
          pl.BlockSpec((_BM, N), lambda i: (i, 0)),
      ],
      out_shape=[
          jax.ShapeDtypeStruct((N_NODES, N), jnp.float32),
          jax.ShapeDtypeStruct((N_NODES, N), jnp.float32),
      ],
  )(s_pair, s_pair, cnt_t, z, WlT, WrT, b)


def _tc_combine_mlp(s_pair, cnt_t, z, W1T, b1, W2T, b2, W3T, b3):
  def body(sa_ref, sb_ref, c_ref, z_ref, w1_ref, b1_ref,
           w2_ref, b2_ref, w3_ref, b3_ref, o_ref):
    ssum = sa_ref[0] + sb_ref[0]
    cnt = jnp.sum(c_ref[...], axis=1, keepdims=True)
    h = ssum / jnp.maximum(cnt, 1.0) + z_ref[...]
    a = jnp.maximum(
        jnp.dot(h, w1_ref[...], preferred_element_type=jnp.float32,
                     precision=lax.Precision.HIGHEST)
        + b1_ref[...], 0.0)
    a = jnp.maximum(
        jnp.dot(a, w2_ref[...], preferred_element_type=jnp.float32,
                     precision=lax.Precision.HIGHEST)
        + b2_ref[...], 0.0)
    o_ref[...] = jnp.sum(a * w3_ref[...], axis=1, keepdims=True) + b3_ref[...]

  return pl.pallas_call(
      body,
      grid=(N_NODES // _BM,),
      in_specs=[
          pl.BlockSpec((1, _BM, HID), lambda i: (0, i, 0)),
          pl.BlockSpec((1, _BM, HID), lambda i: (1, i, 0)),
          pl.BlockSpec((_BM, N_TILES), lambda i: (i, 0)),
          pl.BlockSpec((_BM, HID), lambda i: (i, 0)),
          pl.BlockSpec((HID, 64), lambda i: (0, 0)),
          pl.BlockSpec((1, 64), lambda i: (0, 0)),
          pl.BlockSpec((64, 32), lambda i: (0, 0)),
          pl.BlockSpec((1, 32), lambda i: (0, 0)),
          pl.BlockSpec((1, 32), lambda i: (0, 0)),
          pl.BlockSpec((1, 1), lambda i: (0, 0)),
      ],
      out_specs=pl.BlockSpec((_BM, 1), lambda i: (i, 0)),
      out_shape=jax.ShapeDtypeStruct((N_NODES, 1), jnp.float32),
  )(s_pair, s_pair, cnt_t, z, W1T, b1, W2T, b2, W3T, b3)


def kernel(x, edge_index, W1l, W1r, b1, W2l, W2r, b2, Wr1, br1, Wr2, br2,
           Wr3, br3):
  f32 = jnp.float32

  xp = x.astype(f32)

  src = edge_index[0].astype(jnp.int32)
  dst = edge_index[1].astype(jnp.int32)
  n_e = src.shape[0]
  src = jnp.arange(EP, dtype=jnp.int32) % N_NODES
  src = src.at[:n_e].set(edge_index[0].astype(jnp.int32))
  dst = jnp.full((EP,), N_NODES, jnp.int32).at[:n_e].set(dst)
  src = src.reshape(N_CROWS, CHUNK)
  dst = dst.reshape(N_CROWS, CHUNK)

  zeros_f = jnp.zeros((NP, HID), f32)

  y1, z1 = _tc_proj2(xp, W1l.T.astype(f32), W1r.T.astype(f32),
                     b1.reshape(1, HID).astype(f32))
  s1, cnt_h = _segsum_cnt(y1, src, dst, zeros_f)
  cnt = cnt_h.T

  y2, z2 = _tc_combine_proj2(s1, cnt, z1, W2l.T.astype(f32),
                             W2r.T.astype(f32), b2.reshape(1, HID).astype(f32))

  (s2,) = _segsum(y2, src, dst, zeros_f)
  out = _tc_combine_mlp(s2, cnt, z2,
                        Wr1.T.astype(f32), br1.reshape(1, 64).astype(f32),
                        Wr2.T.astype(f32), br2.reshape(1, 32).astype(f32),
                        Wr3.astype(f32), br3.reshape(1, 1).astype(f32))
  return out[:, 0]


# speedup vs baseline: 1.1072x; 1.1072x over previous
import functools

import jax
import jax.numpy as jnp
from jax import lax
from jax.experimental import pallas as pl
from jax.experimental.pallas import tpu as pltpu
from jax.experimental.pallas import tpu_sc as plsc

N_NODES = 10000
IN_CH = 256
HID = 128

NP = 10240
N_TILES = 32
N_CHUNKS = 40
CHUNK = 128
EP = N_TILES * N_CHUNKS * CHUNK
N_CROWS = EP // CHUNK
SEG_C0 = 40
SEG_C1 = 40
IDX_ROWS = 1280
ROWS_PER_TILE = NP // 16
N_CNT = N_TILES


def _make_segsum(with_cnt):
  mesh = plsc.VectorSubcoreMesh(core_axis_name="c", subcore_axis_name="s")

  dst_rows = 8 if with_cnt else SEG_C0
  out_type = [jax.ShapeDtypeStruct((2, NP, HID), jnp.float32)]
  scratch = [
      pltpu.VMEM((SEG_C0, CHUNK), jnp.int32),
      pltpu.VMEM((dst_rows, CHUNK), jnp.int32),
      pltpu.VMEM((CHUNK, HID), jnp.float32),
      pltpu.VMEM((CHUNK, HID), jnp.float32),
      pltpu.VMEM_SHARED((NP, HID), jnp.float32),
      pltpu.SemaphoreType.DMA,
      pltpu.SemaphoreType.DMA,
  ]
  if with_cnt:
    out_type.append(jax.ShapeDtypeStruct((N_TILES, NP), jnp.float32))
    scratch += [
        pltpu.VMEM((NP,), jnp.float32),
    ]

  def body(*refs):
    if with_cnt:
      (y_hbm, srci_hbm, dsti_hbm, zf_hbm, s_out, cnt_out,
       srci_v, dsti_v, rows0_v, rows1_v, acc_sh, sem0, sem1, hist_v) = refs
    else:
      (y_hbm, srci_hbm, dsti_hbm, zf_hbm, s_out,
       srci_v, dsti_v, rows0_v, rows1_v, acc_sh, sem0, sem1) = refs
    bufs = ((rows0_v, sem0), (rows1_v, sem1))

    c = lax.axis_index("c")
    s = lax.axis_index("s")
    r0 = s * ROWS_PER_TILE
    off = jnp.where(c == 0, s * SEG_C0, 16 * SEG_C0 + s * SEG_C1)
    n = jnp.where(c == 0, SEG_C0, SEG_C1)

    pltpu.sync_copy(zf_hbm.at[pl.ds(r0, ROWS_PER_TILE)],
                    acc_sh.at[pl.ds(r0, ROWS_PER_TILE)])
    ones16 = jnp.full((16,), 1.0, jnp.float32)
    zeros16 = jnp.zeros((16,), jnp.float32)
    if with_cnt:
      def zero_row(i, carry):
        for k in range(8):
          hist_v[pl.ds(i * 128 + k * 16, 16)] = zeros16
        return carry

      lax.fori_loop(0, NP // 128, zero_row, 0)
    plsc.subcore_barrier()

    pltpu.sync_copy(srci_hbm.at[pl.ds(off, SEG_C0)], srci_v)
    if not with_cnt:
      pltpu.sync_copy(dsti_hbm.at[pl.ds(off, SEG_C0)], dsti_v)

    for k, (buf, sem) in enumerate(bufs):
      for h in (0, 64):
        pltpu.async_copy(y_hbm.at[srci_v.at[k, pl.ds(h, 64)]],
                         buf.at[pl.ds(h, 64)], sem)

    if with_cnt:
      def window_step(w, carry):
        pltpu.sync_copy(dsti_hbm.at[pl.ds(off + w * 8, 8)], dsti_v)
        for jj in range(8):
          buf, sem = bufs[jj % 2]
          j = w * 8 + jj
          for h in (0, 64):
            pltpu.make_async_copy(y_hbm.at[srci_v.at[j, pl.ds(h, 64)]],
                                  buf.at[pl.ds(h, 64)], sem).wait()
          pltpu.sync_copy(buf, acc_sh.at[dsti_v.at[jj]], add=True)
          nxt = jnp.minimum(j + 2, n - 1)
          for h in (0, 64):
            pltpu.async_copy(y_hbm.at[srci_v.at[nxt, pl.ds(h, 64)]],
                             buf.at[pl.ds(h, 64)], sem)
          for k8 in range(8):
            d = dsti_v[jj, pl.ds(k8 * 16, 16)]
            plsc.addupdate_scatter(hist_v, [d], ones16)
        return carry

      lax.fori_loop(0, n // 8, window_step, 0)
    else:
      def chunk_step(i, carry):
        for k, (buf, sem) in enumerate(bufs):
          j = 2 * i + k
          for h in (0, 64):
            pltpu.make_async_copy(y_hbm.at[srci_v.at[j, pl.ds(h, 64)]],
                                  buf.at[pl.ds(h, 64)], sem).wait()
          pltpu.sync_copy(buf, acc_sh.at[dsti_v.at[j]], add=True)
          nxt = jnp.minimum(j + 2, n - 1)
          for h in (0, 64):
            pltpu.async_copy(y_hbm.at[srci_v.at[nxt, pl.ds(h, 64)]],
                             buf.at[pl.ds(h, 64)], sem)
        return carry

      lax.fori_loop(0, n // 2, chunk_step, 0)
    for buf, sem in bufs:
      for h in (0, 64):
        pltpu.make_async_copy(y_hbm.at[srci_v.at[0, pl.ds(h, 64)]],
                              buf.at[pl.ds(h, 64)], sem).wait()
    plsc.subcore_barrier()

    pltpu.sync_copy(acc_sh.at[pl.ds(r0, ROWS_PER_TILE)],
                    s_out.at[c, pl.ds(r0, ROWS_PER_TILE)])
    if with_cnt:
      pltpu.sync_copy(hist_v, cnt_out.at[c * 16 + s])

  params = pltpu.CompilerParams(needs_layout_passes=False) if with_cnt else None
  return pl.kernel(body, mesh=mesh, out_type=out_type, scratch_types=scratch,
                   compiler_params=params)


_segsum_cnt = _make_segsum(True)
_segsum = _make_segsum(False)


_BM = 2000


def _tc_proj2(xp, WlT, WrT, b):
  M, K = xp.shape
  N = WlT.shape[1]

  def body(x_ref, wl_ref, wr_ref, b_ref, y_ref, z_ref):
    x = x_ref[...]
    y_ref[...] = jnp.dot(x, wl_ref[...], preferred_element_type=jnp.float32)
    z_ref[...] = (jnp.dot(x, wr_ref[...], preferred_element_type=jnp.float32)
                  + b_ref[...])

  return pl.pallas_call(
      body,
      grid=(M // _BM,),
      in_specs=[
          pl.BlockSpec((_BM, K), lambda i: (i, 0)),
          pl.BlockSpec((K, N), lambda i: (0, 0)),
          pl.BlockSpec((K, N), lambda i: (0, 0)),
          pl.BlockSpec((1, N), lambda i: (0, 0)),
      ],
      out_specs=[
          pl.BlockSpec((_BM, N), lambda i: (i, 0)),
          pl.BlockSpec((_BM, N), lambda i: (i, 0)),
      ],
      out_shape=[
          jax.ShapeDtypeStruct((M, N), jnp.float32),
          jax.ShapeDtypeStruct((M, N), jnp.float32),
      ],
  )(xp, WlT, WrT, b)


def _tc_combine_proj2(s_pair, cnt_t, z, WlT, WrT, b):
  N = WlT.shape[1]

  def body(sa_ref, sb_ref, c_ref, z_ref, wl_ref, wr_ref, b_ref,
           y_ref, z2_ref):
    ssum = sa_ref[0] + sb_ref[0]
    cnt = jnp.sum(c_ref[...], axis=1, keepdims=True)
    mean = ssum / jnp.maximum(cnt, 1.0)
    h = jnp.maximum(mean + z_ref[...], 0.0)
    y_ref[...] = jnp.dot(h, wl_ref[...], preferred_element_type=jnp.float32)
    z2_ref[...] = (jnp.dot(h, wr_ref[...], preferred_element_type=jnp.float32)
                   + b_ref[...])

  return pl.pallas_call(
      body,
      grid=(N_NODES // _BM,),
      in_specs=[
          pl.BlockSpec((1, _BM, HID), lambda i: (0, i, 0)),
          pl.BlockSpec((1, _BM, HID), lambda i: (1, i, 0)),
          pl.BlockSpec((_BM, N_TILES), lambda i: (i, 0)),
          pl.BlockSpec((_BM, HID), lambda i: (i, 0)),
          pl.BlockSpec((HID, N), lambda i: (0, 0)),
          pl.BlockSpec((HID, N), lambda i: (0, 0)),
          pl.BlockSpec((1, N), lambda i: (0, 0)),
      ],
      out_specs=[
          pl.BlockSpec((_BM, N), lambda i: (i, 0)),
          pl.BlockSpec((_BM, N), lambda i: (i, 0)),
      ],
      out_shape=[
          jax.ShapeDtypeStruct((N_NODES, N), jnp.float32),
          jax.ShapeDtypeStruct((N_NODES, N), jnp.float32),
      ],
  )(s_pair, s_pair, cnt_t, z, WlT, WrT, b)


def _tc_combine_mlp(s_pair, cnt_t, z, W1T, b1, W2T, b2, W3T, b3):
  def body(sa_ref, sb_ref, c_ref, z_ref, w1_ref, b1_ref,
           w2_ref, b2_ref, w3_ref, b3_ref, o_ref):
    ssum = sa_ref[0] + sb_ref[0]
    cnt = jnp.sum(c_ref[...], axis=1, keepdims=True)
    h = ssum / jnp.maximum(cnt, 1.0) + z_ref[...]
    a = jnp.maximum(
        jnp.dot(h, w1_ref[...], preferred_element_type=jnp.float32)
        + b1_ref[...], 0.0)
    a = jnp.maximum(
        jnp.dot(a, w2_ref[...], preferred_element_type=jnp.float32)
        + b2_ref[...], 0.0)
    o_ref[...] = jnp.sum(a * w3_ref[...], axis=1, keepdims=True) + b3_ref[...]

  return pl.pallas_call(
      body,
      grid=(N_NODES // _BM,),
      in_specs=[
          pl.BlockSpec((1, _BM, HID), lambda i: (0, i, 0)),
          pl.BlockSpec((1, _BM, HID), lambda i: (1, i, 0)),
          pl.BlockSpec((_BM, N_TILES), lambda i: (i, 0)),
          pl.BlockSpec((_BM, HID), lambda i: (i, 0)),
          pl.BlockSpec((HID, 64), lambda i: (0, 0)),
          pl.BlockSpec((1, 64), lambda i: (0, 0)),
          pl.BlockSpec((64, 32), lambda i: (0, 0)),
          pl.BlockSpec((1, 32), lambda i: (0, 0)),
          pl.BlockSpec((1, 32), lambda i: (0, 0)),
          pl.BlockSpec((1, 1), lambda i: (0, 0)),
      ],
      out_specs=pl.BlockSpec((_BM, 1), lambda i: (i, 0)),
      out_shape=jax.ShapeDtypeStruct((N_NODES, 1), jnp.float32),
  )(s_pair, s_pair, cnt_t, z, W1T, b1, W2T, b2, W3T, b3)


def kernel(x, edge_index, W1l, W1r, b1, W2l, W2r, b2, Wr1, br1, Wr2, br2,
           Wr3, br3):
  f32 = jnp.float32

  xp = x.astype(f32)

  src = edge_index[0].astype(jnp.int32)
  dst = edge_index[1].astype(jnp.int32)
  n_e = src.shape[0]
  src = jnp.arange(EP, dtype=jnp.int32) % N_NODES
  src = src.at[:n_e].set(edge_index[0].astype(jnp.int32))
  dst = jnp.full((EP,), N_NODES, jnp.int32).at[:n_e].set(dst)
  src = src.reshape(N_CROWS, CHUNK)
  dst = dst.reshape(N_CROWS, CHUNK)

  zeros_f = jnp.zeros((NP, HID), f32)

  y1, z1 = _tc_proj2(xp, W1l.T.astype(f32), W1r.T.astype(f32),
                     b1.reshape(1, HID).astype(f32))
  s1, cnt_h = _segsum_cnt(y1, src, dst, zeros_f)
  cnt = cnt_h.T

  y2, z2 = _tc_combine_proj2(s1, cnt, z1, W2l.T.astype(f32),
                             W2r.T.astype(f32), b2.reshape(1, HID).astype(f32))

  (s2,) = _segsum(y2, src, dst, zeros_f)
  out = _tc_combine_mlp(s2, cnt, z2,
                        Wr1.T.astype(f32), br1.reshape(1, 64).astype(f32),
                        Wr2.T.astype(f32), br2.reshape(1, 32).astype(f32),
                        Wr3.astype(f32), br3.reshape(1, 1).astype(f32))
  return out[:, 0]


# speedup vs baseline: 1.1109x; 1.0034x over previous
import jax
import jax.numpy as jnp
from jax import lax
from jax.experimental import pallas as pl
from jax.experimental.pallas import tpu as pltpu
from jax.experimental.pallas import tpu_sc as plsc

N_NODES = 10000
IN_CH = 256
HID = 128

NP = 10240
N_TILES = 32
N_CHUNKS = 40
CHUNK = 128
EP = N_TILES * N_CHUNKS * CHUNK
N_CROWS = EP // CHUNK
SEG_C0 = 40
SEG_C1 = 40
ROWS_PER_TILE = NP // 16


def _make_segsum(with_cnt):
  mesh = plsc.VectorSubcoreMesh(core_axis_name="c", subcore_axis_name="s")

  dst_rows = 8 if with_cnt else SEG_C0
  out_type = [jax.ShapeDtypeStruct((2, NP, HID), jnp.float32)]
  scratch = [
      pltpu.VMEM((SEG_C0, CHUNK), jnp.int32),
      pltpu.VMEM((dst_rows, CHUNK), jnp.int32),
      pltpu.VMEM((CHUNK, HID), jnp.float32),
      pltpu.VMEM((CHUNK, HID), jnp.float32),
      pltpu.VMEM_SHARED((NP, HID), jnp.float32),
      pltpu.SemaphoreType.DMA,
      pltpu.SemaphoreType.DMA,
  ]
  if with_cnt:
    out_type.append(jax.ShapeDtypeStruct((N_TILES, NP), jnp.float32))
    scratch += [
        pltpu.VMEM((NP,), jnp.float32),
    ]

  def body(*refs):
    if with_cnt:
      (y_hbm, srci_hbm, dsti_hbm, zf_hbm, s_out, cnt_out,
       srci_v, dsti_v, rows0_v, rows1_v, acc_sh, sem0, sem1, hist_v) = refs
    else:
      (y_hbm, srci_hbm, dsti_hbm, zf_hbm, s_out,
       srci_v, dsti_v, rows0_v, rows1_v, acc_sh, sem0, sem1) = refs
    bufs = ((rows0_v, sem0), (rows1_v, sem1))

    c = lax.axis_index("c")
    s = lax.axis_index("s")
    r0 = s * ROWS_PER_TILE
    off = jnp.where(c == 0, s * SEG_C0, 16 * SEG_C0 + s * SEG_C1)
    n = jnp.where(c == 0, SEG_C0, SEG_C1)

    pltpu.sync_copy(zf_hbm.at[pl.ds(r0, ROWS_PER_TILE)],
                    acc_sh.at[pl.ds(r0, ROWS_PER_TILE)])
    ones16 = jnp.full((16,), 1.0, jnp.float32)
    zeros16 = jnp.zeros((16,), jnp.float32)
    if with_cnt:
      def zero_row(i, carry):
        for k in range(8):
          hist_v[pl.ds(i * 128 + k * 16, 16)] = zeros16
        return carry

      lax.fori_loop(0, NP // 128, zero_row, 0)
    plsc.subcore_barrier()

    pltpu.sync_copy(srci_hbm.at[pl.ds(off, SEG_C0)], srci_v)
    if not with_cnt:
      pltpu.sync_copy(dsti_hbm.at[pl.ds(off, SEG_C0)], dsti_v)

    for k, (buf, sem) in enumerate(bufs):
      pltpu.async_copy(y_hbm.at[srci_v.at[k]], buf, sem)

    if with_cnt:
      def window_step(w, carry):
        pltpu.sync_copy(dsti_hbm.at[pl.ds(off + w * 8, 8)], dsti_v)
        for jj in range(8):
          buf, sem = bufs[jj % 2]
          j = w * 8 + jj
          pltpu.make_async_copy(y_hbm.at[srci_v.at[j]], buf, sem).wait()
          pltpu.sync_copy(buf, acc_sh.at[dsti_v.at[jj]], add=True)
          nxt = jnp.minimum(j + 2, n - 1)
          pltpu.async_copy(y_hbm.at[srci_v.at[nxt]], buf, sem)
          for k8 in range(8):
            d = dsti_v[jj, pl.ds(k8 * 16, 16)]
            plsc.addupdate_scatter(hist_v, [d], ones16)
        return carry

      lax.fori_loop(0, n // 8, window_step, 0)
    else:
      def chunk_step(i, carry):
        for k, (buf, sem) in enumerate(bufs):
          j = 2 * i + k
          pltpu.make_async_copy(y_hbm.at[srci_v.at[j]], buf, sem).wait()
          pltpu.sync_copy(buf, acc_sh.at[dsti_v.at[j]], add=True)
          nxt = jnp.minimum(j + 2, n - 1)
          pltpu.async_copy(y_hbm.at[srci_v.at[nxt]], buf, sem)
        return carry

      lax.fori_loop(0, n // 2, chunk_step, 0)
    for buf, sem in bufs:
      pltpu.make_async_copy(y_hbm.at[srci_v.at[0]], buf, sem).wait()
    plsc.subcore_barrier()

    pltpu.sync_copy(acc_sh.at[pl.ds(r0, ROWS_PER_TILE)],
                    s_out.at[c, pl.ds(r0, ROWS_PER_TILE)])
    if with_cnt:
      pltpu.sync_copy(hist_v, cnt_out.at[c * 16 + s])

  params = pltpu.CompilerParams(needs_layout_passes=False) if with_cnt else None
  return pl.kernel(body, mesh=mesh, out_type=out_type, scratch_types=scratch,
                   compiler_params=params)


_segsum_cnt = _make_segsum(True)
_segsum = _make_segsum(False)


_BM = 2000


def _tc_proj2(xp, WlT, WrT, b):
  M, K = xp.shape
  N = WlT.shape[1]

  def body(x_ref, wl_ref, wr_ref, b_ref, y_ref, z_ref):
    x = x_ref[...]
    y_ref[...] = jnp.dot(x, wl_ref[...], preferred_element_type=jnp.float32)
    z_ref[...] = (jnp.dot(x, wr_ref[...], preferred_element_type=jnp.float32)
                  + b_ref[...])

  return pl.pallas_call(
      body,
      grid=(M // _BM,),
      in_specs=[
          pl.BlockSpec((_BM, K), lambda i: (i, 0)),
          pl.BlockSpec((K, N), lambda i: (0, 0)),
          pl.BlockSpec((K, N), lambda i: (0, 0)),
          pl.BlockSpec((1, N), lambda i: (0, 0)),
      ],
      out_specs=[
          pl.BlockSpec((_BM, N), lambda i: (i, 0)),
          pl.BlockSpec((_BM, N), lambda i: (i, 0)),
      ],
      out_shape=[
          jax.ShapeDtypeStruct((M, N), jnp.float32),
          jax.ShapeDtypeStruct((M, N), jnp.float32),
      ],
  )(xp, WlT, WrT, b)


def _tc_combine_proj2(s_pair, cnt_t, z, WlT, WrT, b):
  N = WlT.shape[1]

  def body(sa_ref, sb_ref, c_ref, z_ref, wl_ref, wr_ref, b_ref,
           y_ref, z2_ref):
    ssum = sa_ref[0] + sb_ref[0]
    cnt = jnp.sum(c_ref[...], axis=1, keepdims=True)
    mean = ssum / jnp.maximum(cnt, 1.0)
    h = jnp.maximum(mean + z_ref[...], 0.0)
    y_ref[...] = jnp.dot(h, wl_ref[...], preferred_element_type=jnp.float32)
    z2_ref[...] = (jnp.dot(h, wr_ref[...], preferred_element_type=jnp.float32)
                   + b_ref[...])

  return pl.pallas_call(
      body,
      grid=(N_NODES // _BM,),
      in_specs=[
          pl.BlockSpec((1, _BM, HID), lambda i: (0, i, 0)),
          pl.BlockSpec((1, _BM, HID), lambda i: (1, i, 0)),
          pl.BlockSpec((_BM, N_TILES), lambda i: (i, 0)),
          pl.BlockSpec((_BM, HID), lambda i: (i, 0)),
          pl.BlockSpec((HID, N), lambda i: (0, 0)),
          pl.BlockSpec((HID, N), lambda i: (0, 0)),
          pl.BlockSpec((1, N), lambda i: (0, 0)),
      ],
      out_specs=[
          pl.BlockSpec((_BM, N), lambda i: (i, 0)),
          pl.BlockSpec((_BM, N), lambda i: (i, 0)),
      ],
      out_shape=[
          jax.ShapeDtypeStruct((N_NODES, N), jnp.float32),
          jax.ShapeDtypeStruct((N_NODES, N), jnp.float32),
      ],
  )(s_pair, s_pair, cnt_t, z, WlT, WrT, b)


def _tc_combine_mlp(s_pair, cnt_t, z, W1T, b1, W2T, b2, W3T, b3):
  def body(sa_ref, sb_ref, c_ref, z_ref, w1_ref, b1_ref,
           w2_ref, b2_ref, w3_ref, b3_ref, o_ref):
    ssum = sa_ref[0] + sb_ref[0]
    cnt = jnp.sum(c_ref[...], axis=1, keepdims=True)
    h = ssum / jnp.maximum(cnt, 1.0) + z_ref[...]
    a = jnp.maximum(
        jnp.dot(h, w1_ref[...], preferred_element_type=jnp.float32)
        + b1_ref[...], 0.0)
    a = jnp.maximum(
        jnp.dot(a, w2_ref[...], preferred_element_type=jnp.float32)
        + b2_ref[...], 0.0)
    o_ref[...] = jnp.sum(a * w3_ref[...], axis=1, keepdims=True) + b3_ref[...]

  return pl.pallas_call(
      body,
      grid=(N_NODES // _BM,),
      in_specs=[
          pl.BlockSpec((1, _BM, HID), lambda i: (0, i, 0)),
          pl.BlockSpec((1, _BM, HID), lambda i: (1, i, 0)),
          pl.BlockSpec((_BM, N_TILES), lambda i: (i, 0)),
          pl.BlockSpec((_BM, HID), lambda i: (i, 0)),
          pl.BlockSpec((HID, 64), lambda i: (0, 0)),
          pl.BlockSpec((1, 64), lambda i: (0, 0)),
          pl.BlockSpec((64, 32), lambda i: (0, 0)),
          pl.BlockSpec((1, 32), lambda i: (0, 0)),
          pl.BlockSpec((1, 32), lambda i: (0, 0)),
          pl.BlockSpec((1, 1), lambda i: (0, 0)),
      ],
      out_specs=pl.BlockSpec((_BM, 1), lambda i: (i, 0)),
      out_shape=jax.ShapeDtypeStruct((N_NODES, 1), jnp.float32),
  )(s_pair, s_pair, cnt_t, z, W1T, b1, W2T, b2, W3T, b3)


def kernel(x, edge_index, W1l, W1r, b1, W2l, W2r, b2, Wr1, br1, Wr2, br2,
           Wr3, br3):
  f32 = jnp.float32

  xp = x.astype(f32)

  src = edge_index[0].astype(jnp.int32)
  dst = edge_index[1].astype(jnp.int32)
  n_e = src.shape[0]
  src = jnp.arange(EP, dtype=jnp.int32) % N_NODES
  src = src.at[:n_e].set(edge_index[0].astype(jnp.int32))
  dst = jnp.full((EP,), N_NODES, jnp.int32).at[:n_e].set(dst)
  src = src.reshape(N_CROWS, CHUNK)
  dst = dst.reshape(N_CROWS, CHUNK)

  zeros_f = jnp.zeros((NP, HID), f32)

  y1, z1 = _tc_proj2(xp, W1l.T.astype(f32), W1r.T.astype(f32),
                     b1.reshape(1, HID).astype(f32))
  s1, cnt_h = _segsum_cnt(y1, src, dst, zeros_f)
  cnt = cnt_h.T

  y2, z2 = _tc_combine_proj2(s1, cnt, z1, W2l.T.astype(f32),
                             W2r.T.astype(f32), b2.reshape(1, HID).astype(f32))

  (s2,) = _segsum(y2, src, dst, zeros_f)
  out = _tc_combine_mlp(s2, cnt, z2,
                        Wr1.T.astype(f32), br1.reshape(1, 64).astype(f32),
                        Wr2.T.astype(f32), br2.reshape(1, 32).astype(f32),
                        Wr3.astype(f32), br3.reshape(1, 1).astype(f32))
  return out[:, 0]
